# Initial kernel scaffold; baseline (speedup 1.0000x reference)
import jax
import jax.numpy as jnp
from jax.experimental import pallas as pl


def kernel(x, edge_index, W1, b1, W2, b2):
    raise NotImplementedError("write your pallas kernel here")


# speedup vs baseline: 26.5442x; 26.5442x over previous
import functools

import jax
import jax.numpy as jnp
from jax import lax
from jax.experimental import pallas as pl
from jax.experimental.pallas import tpu as pltpu
from jax.experimental.pallas import tpu_sc as plsc

N = 10000
E = 320000
D_IN = 128
H = 16
C = 40

NC = 2
NS = 16
NW = NC * NS
EPT = E // NW
CH = 80
NCH = EPT // CH
RPT = 632
RPT_LAST = N - 15 * RPT

_MESH = plsc.VectorSubcoreMesh(core_axis_name="c", subcore_axis_name="s")
_SC_PARAMS = pltpu.CompilerParams(use_tc_tiling_on_sc=False)


@functools.partial(
    pl.kernel,
    out_type=jax.ShapeDtypeStruct((NC * N,), jnp.float32),
    mesh=_MESH,
    compiler_params=_SC_PARAMS,
    scratch_types=[
        pltpu.VMEM((NCH, CH), jnp.int32),
        pltpu.VMEM((CH,), jnp.float32),
        pltpu.VMEM((1008,), jnp.float32),
        pltpu.VMEM_SHARED((N,), jnp.float32),
        pltpu.SemaphoreType.DMA,
    ],
)
def _deg_kernel(dst_hbm, out_hbm, idx_v, ones_v, stage_v, deg_sh, sem):
    cid = lax.axis_index("c")
    sid = lax.axis_index("s")
    wid = cid * NS + sid

    pltpu.sync_copy(dst_hbm.at[wid], idx_v)

    def fill_ones(i, _):
        ones_v[pl.ds(i * 16, 16)] = jnp.ones((16,), jnp.float32)
        return ()
    lax.fori_loop(0, CH // 16, fill_ones, ())

    def fill_zeros(i, _):
        stage_v[pl.ds(i * 16, 16)] = jnp.zeros((16,), jnp.float32)
        return ()
    lax.fori_loop(0, 63, fill_zeros, ())

    @pl.when(sid < 10)
    def _():
        pltpu.sync_copy(stage_v.at[pl.ds(0, 1000)],
                        deg_sh.at[pl.ds(sid * 1000, 1000)])

    plsc.subcore_barrier()

    def body(j, _):
        pltpu.sync_copy(ones_v, deg_sh.at[idx_v.at[j]], add=True)
        return ()
    lax.fori_loop(0, NCH, body, ())

    plsc.subcore_barrier()

    @pl.when(sid < 10)
    def _():
        pltpu.sync_copy(deg_sh.at[pl.ds(sid * 1000, 1000)],
                        stage_v.at[pl.ds(0, 1000)])
        pltpu.sync_copy(stage_v.at[pl.ds(0, 1000)],
                        out_hbm.at[pl.ds(cid * N + sid * 1000, 1000)])


def _make_agg_kernel(feat):
    @functools.partial(
        pl.kernel,
        out_type=jax.ShapeDtypeStruct((NC, N, feat), jnp.float32),
        mesh=_MESH,
        compiler_params=_SC_PARAMS,
        scratch_types=[
            pltpu.VMEM((NCH, CH), jnp.int32),
            pltpu.VMEM((NCH, CH), jnp.int32),
            pltpu.VMEM((CH, feat), jnp.float32),
            pltpu.VMEM((RPT, feat), jnp.float32),
            pltpu.VMEM_SHARED((N, feat), jnp.float32),
            pltpu.SemaphoreType.DMA,
        ],
    )
    def _agg(h_hbm, src_hbm, dst_hbm, out_hbm,
             src_v, dst_v, rows_v, stage_v, acc_sh, sem):
        cid = lax.axis_index("c")
        sid = lax.axis_index("s")
        wid = cid * NS + sid

        pltpu.sync_copy(src_hbm.at[wid], src_v)
        pltpu.sync_copy(dst_hbm.at[wid], dst_v)
        @pl.when(sid < 15)
        def _():
            pltpu.sync_copy(h_hbm.at[pl.ds(sid * RPT, RPT)], stage_v)
            pltpu.sync_copy(stage_v, acc_sh.at[pl.ds(sid * RPT, RPT)])

        @pl.when(sid == 15)
        def _():
            pltpu.sync_copy(h_hbm.at[pl.ds(15 * RPT, RPT_LAST)],
                            stage_v.at[pl.ds(0, RPT_LAST)])
            pltpu.sync_copy(stage_v.at[pl.ds(0, RPT_LAST)],
                            acc_sh.at[pl.ds(15 * RPT, RPT_LAST)])

        plsc.subcore_barrier()

        def body(j, _):
            pltpu.async_copy(h_hbm.at[src_v.at[j]], rows_v, sem).wait()
            pltpu.sync_copy(rows_v, acc_sh.at[dst_v.at[j]], add=True)
            return ()
        lax.fori_loop(0, NCH, body, ())

        plsc.subcore_barrier()

        @pl.when(sid < 15)
        def _():
            pltpu.sync_copy(acc_sh.at[pl.ds(sid * RPT, RPT)], stage_v)
            pltpu.sync_copy(stage_v,
                            out_hbm.at[cid, pl.ds(sid * RPT, RPT)])

        @pl.when(sid == 15)
        def _():
            pltpu.sync_copy(acc_sh.at[pl.ds(15 * RPT, RPT_LAST)],
                            stage_v.at[pl.ds(0, RPT_LAST)])
            pltpu.sync_copy(stage_v.at[pl.ds(0, RPT_LAST)],
                            out_hbm.at[cid, pl.ds(15 * RPT, RPT_LAST)])

    return _agg


_agg16 = _make_agg_kernel(H)
_agg40 = _make_agg_kernel(C)


_RB = 1000
_GRID = N // _RB


def _tc1_body(x_ref, w1_ref, degp_ref, h1p_ref, dis_ref):
    deg = degp_ref[0] + degp_ref[1] + 1.0
    dis = lax.rsqrt(deg)
    h = jnp.dot(x_ref[...], w1_ref[...], preferred_element_type=jnp.float32)
    h1p_ref[...] = h * dis
    dis_ref[...] = dis


def _tc1(x, w1, degp):
    return pl.pallas_call(
        _tc1_body,
        grid=(_GRID,),
        in_specs=[
            pl.BlockSpec((_RB, D_IN), lambda i: (i, 0)),
            pl.BlockSpec((D_IN, H), lambda i: (0, 0)),
            pl.BlockSpec((NC, _RB, 1), lambda i: (0, i, 0)),
        ],
        out_specs=[
            pl.BlockSpec((_RB, H), lambda i: (i, 0)),
            pl.BlockSpec((_RB, 1), lambda i: (i, 0)),
        ],
        out_shape=[
            jax.ShapeDtypeStruct((N, H), jnp.float32),
            jax.ShapeDtypeStruct((N, 1), jnp.float32),
        ],
    )(x, w1, degp)


def _tc2_body(p_ref, h1p_ref, dis_ref, b1_ref, w2_ref, h2p_ref):
    dis = dis_ref[...]
    agg = p_ref[0] + p_ref[1] - h1p_ref[...]
    a1 = jnp.maximum(agg * dis + b1_ref[...], 0.0)
    h2 = jnp.dot(a1, w2_ref[...], preferred_element_type=jnp.float32)
    h2p_ref[...] = h2 * dis


def _tc2(p, h1p, dis, b1, w2):
    return pl.pallas_call(
        _tc2_body,
        grid=(_GRID,),
        in_specs=[
            pl.BlockSpec((NC, _RB, H), lambda i: (0, i, 0)),
            pl.BlockSpec((_RB, H), lambda i: (i, 0)),
            pl.BlockSpec((_RB, 1), lambda i: (i, 0)),
            pl.BlockSpec((1, H), lambda i: (0, 0)),
            pl.BlockSpec((H, C), lambda i: (0, 0)),
        ],
        out_specs=pl.BlockSpec((_RB, C), lambda i: (i, 0)),
        out_shape=jax.ShapeDtypeStruct((N, C), jnp.float32),
    )(p, h1p, dis, b1, w2)


def _tc3_body(q_ref, h2p_ref, dis_ref, b2_ref, out_ref):
    z = (q_ref[0] + q_ref[1] - h2p_ref[...]) * dis_ref[...] + b2_ref[...]
    m = jnp.max(z, axis=1, keepdims=True)
    s = jnp.sum(jnp.exp(z - m), axis=1, keepdims=True)
    out_ref[...] = z - m - jnp.log(s)


def _tc3(q, h2p, dis, b2):
    return pl.pallas_call(
        _tc3_body,
        grid=(_GRID,),
        in_specs=[
            pl.BlockSpec((NC, _RB, C), lambda i: (0, i, 0)),
            pl.BlockSpec((_RB, C), lambda i: (i, 0)),
            pl.BlockSpec((_RB, 1), lambda i: (i, 0)),
            pl.BlockSpec((1, C), lambda i: (0, 0)),
        ],
        out_specs=pl.BlockSpec((_RB, C), lambda i: (i, 0)),
        out_shape=jax.ShapeDtypeStruct((N, C), jnp.float32),
    )(q, h2p, dis, b2)


@jax.jit
def kernel(x, edge_index, W1, b1, W2, b2):
    src = edge_index[0].reshape(NW, NCH, CH)
    dst = edge_index[1].reshape(NW, NCH, CH)

    degp = _deg_kernel(dst)
    h1p, dis = _tc1(x, W1, degp.reshape(NC, N, 1))
    p = _agg16(h1p, src, dst)
    h2p = _tc2(p, h1p, dis, b1.reshape(1, H), W2)
    q = _agg40(h2p, src, dst)
    return _tc3(q, h2p, dis, b2.reshape(1, C))


# speedup vs baseline: 37.4215x; 1.4098x over previous
import functools

import jax
import jax.numpy as jnp
from jax import lax
from jax.experimental import pallas as pl
from jax.experimental.pallas import tpu as pltpu
from jax.experimental.pallas import tpu_sc as plsc

N = 10000
E = 320000
D_IN = 128
H = 16
C = 40

NC = 2
NS = 16
NW = NC * NS
EPT = E // NW
CH = 80
NCH = EPT // CH
RPT = 632
RPT_LAST = N - 15 * RPT

_MESH = plsc.VectorSubcoreMesh(core_axis_name="c", subcore_axis_name="s")
_SC_PARAMS = pltpu.CompilerParams(use_tc_tiling_on_sc=False)


@functools.partial(
    pl.kernel,
    out_type=jax.ShapeDtypeStruct((NC * N,), jnp.float32),
    mesh=_MESH,
    compiler_params=_SC_PARAMS,
    scratch_types=[
        pltpu.VMEM((NCH, CH), jnp.int32),
        pltpu.VMEM((CH,), jnp.float32),
        pltpu.VMEM((1008,), jnp.float32),
        pltpu.VMEM_SHARED((N,), jnp.float32),
        pltpu.SemaphoreType.DMA,
    ],
)
def _deg_kernel(dst_hbm, out_hbm, idx_v, ones_v, stage_v, deg_sh, sem):
    cid = lax.axis_index("c")
    sid = lax.axis_index("s")
    wid = cid * NS + sid

    pltpu.sync_copy(dst_hbm.at[wid], idx_v)

    def fill_ones(i, _):
        ones_v[pl.ds(i * 16, 16)] = jnp.ones((16,), jnp.float32)
        return ()
    lax.fori_loop(0, CH // 16, fill_ones, ())

    def fill_zeros(i, _):
        stage_v[pl.ds(i * 16, 16)] = jnp.zeros((16,), jnp.float32)
        return ()
    lax.fori_loop(0, 63, fill_zeros, ())

    @pl.when(sid < 10)
    def _():
        pltpu.sync_copy(stage_v.at[pl.ds(0, 1000)],
                        deg_sh.at[pl.ds(sid * 1000, 1000)])

    plsc.subcore_barrier()

    def body(j, _):
        pltpu.sync_copy(ones_v, deg_sh.at[idx_v.at[j]], add=True)
        return ()
    lax.fori_loop(0, NCH, body, ())

    plsc.subcore_barrier()

    @pl.when(sid < 10)
    def _():
        pltpu.sync_copy(deg_sh.at[pl.ds(sid * 1000, 1000)],
                        stage_v.at[pl.ds(0, 1000)])
        pltpu.sync_copy(stage_v.at[pl.ds(0, 1000)],
                        out_hbm.at[pl.ds(cid * N + sid * 1000, 1000)])


def _make_agg_kernel(feat):
    @functools.partial(
        pl.kernel,
        out_type=jax.ShapeDtypeStruct((NC, N, feat), jnp.float32),
        mesh=_MESH,
        compiler_params=_SC_PARAMS,
        scratch_types=[
            pltpu.VMEM((NCH, CH), jnp.int32),
            pltpu.VMEM((NCH, CH), jnp.int32),
            pltpu.VMEM((CH, feat), jnp.float32),
            pltpu.VMEM((CH, feat), jnp.float32),
            pltpu.VMEM((RPT, feat), jnp.float32),
            pltpu.VMEM_SHARED((N, feat), jnp.float32),
            pltpu.SemaphoreType.DMA,
            pltpu.SemaphoreType.DMA,
            pltpu.SemaphoreType.DMA,
            pltpu.SemaphoreType.DMA,
        ],
    )
    def _agg(h_hbm, src_hbm, dst_hbm, out_hbm,
             src_v, dst_v, rows0, rows1, stage_v, acc_sh,
             semg0, semg1, sems0, sems1):
        cid = lax.axis_index("c")
        sid = lax.axis_index("s")
        wid = cid * NS + sid

        pltpu.sync_copy(src_hbm.at[wid], src_v)
        pltpu.sync_copy(dst_hbm.at[wid], dst_v)
        @pl.when(sid < 15)
        def _():
            pltpu.sync_copy(h_hbm.at[pl.ds(sid * RPT, RPT)], stage_v)
            pltpu.sync_copy(stage_v, acc_sh.at[pl.ds(sid * RPT, RPT)])

        @pl.when(sid == 15)
        def _():
            pltpu.sync_copy(h_hbm.at[pl.ds(15 * RPT, RPT_LAST)],
                            stage_v.at[pl.ds(0, RPT_LAST)])
            pltpu.sync_copy(stage_v.at[pl.ds(0, RPT_LAST)],
                            acc_sh.at[pl.ds(15 * RPT, RPT_LAST)])

        plsc.subcore_barrier()

        npair = (NCH - 1) // 2

        def _gather(j, buf, sem):
            return pltpu.async_copy(h_hbm.at[src_v.at[j]], buf, sem)

        def _scatter(j, buf, sem):
            return pltpu.async_copy(buf, acc_sh.at[dst_v.at[j]], sem,
                                    add=True)

        def _wait_gather(j, buf, sem):
            pltpu.make_async_copy(h_hbm.at[src_v.at[j]], buf, sem).wait()

        def _wait_scatter(j, buf, sem):
            pltpu.make_async_copy(buf, acc_sh.at[dst_v.at[j]], sem).wait()

        _gather(0, rows0, semg0)

        def body(k, _):
            c0 = 2 * k
            c1 = c0 + 1

            @pl.when(k > 0)
            def _():
                _wait_scatter(c1 - 2, rows1, sems1)
            _gather(c1, rows1, semg1)
            _wait_gather(c0, rows0, semg0)
            _scatter(c0, rows0, sems0)

            @pl.when(k < npair - 1)
            def _():
                _wait_scatter(c0, rows0, sems0)
                _gather(c0 + 2, rows0, semg0)
            _wait_gather(c1, rows1, semg1)
            _scatter(c1, rows1, sems1)
            return ()
        lax.fori_loop(0, npair, body, ())

        _wait_scatter(0, rows0, sems0)
        pltpu.async_copy(h_hbm.at[src_v.at[NCH - 1]], rows0, semg0).wait()
        pltpu.sync_copy(rows0, acc_sh.at[dst_v.at[NCH - 1]], add=True)
        _wait_scatter(1, rows1, sems1)

        plsc.subcore_barrier()

        @pl.when(sid < 15)
        def _():
            pltpu.sync_copy(acc_sh.at[pl.ds(sid * RPT, RPT)], stage_v)
            pltpu.sync_copy(stage_v,
                            out_hbm.at[cid, pl.ds(sid * RPT, RPT)])

        @pl.when(sid == 15)
        def _():
            pltpu.sync_copy(acc_sh.at[pl.ds(15 * RPT, RPT_LAST)],
                            stage_v.at[pl.ds(0, RPT_LAST)])
            pltpu.sync_copy(stage_v.at[pl.ds(0, RPT_LAST)],
                            out_hbm.at[cid, pl.ds(15 * RPT, RPT_LAST)])

    return _agg


_agg16 = _make_agg_kernel(H)
_agg40 = _make_agg_kernel(C)


_RB = 1000
_GRID = N // _RB


def _tc_mm_body(x_ref, w1_ref, h1_ref):
    h1_ref[...] = jnp.dot(x_ref[...], w1_ref[...],
                          preferred_element_type=jnp.float32)


def _tc_mm(x, w1):
    return pl.pallas_call(
        _tc_mm_body,
        grid=(_GRID,),
        in_specs=[
            pl.BlockSpec((_RB, D_IN), lambda i: (i, 0)),
            pl.BlockSpec((D_IN, H), lambda i: (0, 0)),
        ],
        out_specs=pl.BlockSpec((_RB, H), lambda i: (i, 0)),
        out_shape=jax.ShapeDtypeStruct((N, H), jnp.float32),
    )(x, w1)


def _tc1_body(h1_ref, degp_ref, h1p_ref, dis_ref):
    deg = degp_ref[0] + degp_ref[1] + 1.0
    dis = lax.rsqrt(deg)
    h1p_ref[...] = h1_ref[...] * dis
    dis_ref[...] = dis


def _tc1(h1, degp):
    return pl.pallas_call(
        _tc1_body,
        grid=(_GRID,),
        in_specs=[
            pl.BlockSpec((_RB, H), lambda i: (i, 0)),
            pl.BlockSpec((NC, _RB, 1), lambda i: (0, i, 0)),
        ],
        out_specs=[
            pl.BlockSpec((_RB, H), lambda i: (i, 0)),
            pl.BlockSpec((_RB, 1), lambda i: (i, 0)),
        ],
        out_shape=[
            jax.ShapeDtypeStruct((N, H), jnp.float32),
            jax.ShapeDtypeStruct((N, 1), jnp.float32),
        ],
    )(h1, degp)


def _tc2_body(p_ref, h1p_ref, dis_ref, b1_ref, w2_ref, h2p_ref):
    dis = dis_ref[...]
    agg = p_ref[0] + p_ref[1] - h1p_ref[...]
    a1 = jnp.maximum(agg * dis + b1_ref[...], 0.0)
    h2 = jnp.dot(a1, w2_ref[...], preferred_element_type=jnp.float32)
    h2p_ref[...] = h2 * dis


def _tc2(p, h1p, dis, b1, w2):
    return pl.pallas_call(
        _tc2_body,
        grid=(_GRID,),
        in_specs=[
            pl.BlockSpec((NC, _RB, H), lambda i: (0, i, 0)),
            pl.BlockSpec((_RB, H), lambda i: (i, 0)),
            pl.BlockSpec((_RB, 1), lambda i: (i, 0)),
            pl.BlockSpec((1, H), lambda i: (0, 0)),
            pl.BlockSpec((H, C), lambda i: (0, 0)),
        ],
        out_specs=pl.BlockSpec((_RB, C), lambda i: (i, 0)),
        out_shape=jax.ShapeDtypeStruct((N, C), jnp.float32),
    )(p, h1p, dis, b1, w2)


def _tc3_body(q_ref, h2p_ref, dis_ref, b2_ref, out_ref):
    z = (q_ref[0] + q_ref[1] - h2p_ref[...]) * dis_ref[...] + b2_ref[...]
    m = jnp.max(z, axis=1, keepdims=True)
    s = jnp.sum(jnp.exp(z - m), axis=1, keepdims=True)
    out_ref[...] = z - m - jnp.log(s)


def _tc3(q, h2p, dis, b2):
    return pl.pallas_call(
        _tc3_body,
        grid=(_GRID,),
        in_specs=[
            pl.BlockSpec((NC, _RB, C), lambda i: (0, i, 0)),
            pl.BlockSpec((_RB, C), lambda i: (i, 0)),
            pl.BlockSpec((_RB, 1), lambda i: (i, 0)),
            pl.BlockSpec((1, C), lambda i: (0, 0)),
        ],
        out_specs=pl.BlockSpec((_RB, C), lambda i: (i, 0)),
        out_shape=jax.ShapeDtypeStruct((N, C), jnp.float32),
    )(q, h2p, dis, b2)


@jax.jit
def kernel(x, edge_index, W1, b1, W2, b2):
    src = edge_index[0].reshape(NW, NCH, CH)
    dst = edge_index[1].reshape(NW, NCH, CH)

    degp = _deg_kernel(dst)
    h1 = _tc_mm(x, W1)
    h1p, dis = _tc1(h1, degp.reshape(NC, N, 1))
    p = _agg16(h1p, src, dst)
    h2p = _tc2(p, h1p, dis, b1.reshape(1, H), W2)
    q = _agg40(h2p, src, dst)
    return _tc3(q, h2p, dis, b2.reshape(1, C))


# speedup vs baseline: 40.5615x; 1.0839x over previous
import functools

import jax
import jax.numpy as jnp
from jax import lax
from jax.experimental import pallas as pl
from jax.experimental.pallas import tpu as pltpu
from jax.experimental.pallas import tpu_sc as plsc

N = 10000
E = 320000
D_IN = 128
H = 16
C = 40

NC = 2
NS = 16
NW = NC * NS
EPT = E // NW
CH = 80
NCH = EPT // CH
RPT = 632
RPT_LAST = N - 15 * RPT

_MESH = plsc.VectorSubcoreMesh(core_axis_name="c", subcore_axis_name="s")
_SC_PARAMS = pltpu.CompilerParams(use_tc_tiling_on_sc=False)


@functools.partial(
    pl.kernel,
    out_type=jax.ShapeDtypeStruct((NC * N,), jnp.float32),
    mesh=_MESH,
    compiler_params=_SC_PARAMS,
    scratch_types=[
        pltpu.VMEM((NCH, CH), jnp.int32),
        pltpu.VMEM((CH,), jnp.float32),
        pltpu.VMEM((1008,), jnp.float32),
        pltpu.VMEM_SHARED((N,), jnp.float32),
        pltpu.SemaphoreType.DMA,
    ],
)
def _deg_kernel(dst_hbm, out_hbm, idx_v, ones_v, stage_v, deg_sh, sem):
    cid = lax.axis_index("c")
    sid = lax.axis_index("s")
    wid = cid * NS + sid

    pltpu.sync_copy(dst_hbm.at[wid], idx_v)

    def fill_ones(i, _):
        ones_v[pl.ds(i * 16, 16)] = jnp.ones((16,), jnp.float32)
        return ()
    lax.fori_loop(0, CH // 16, fill_ones, ())

    def fill_zeros(i, _):
        stage_v[pl.ds(i * 16, 16)] = jnp.zeros((16,), jnp.float32)
        return ()
    lax.fori_loop(0, 63, fill_zeros, ())

    @pl.when(sid < 10)
    def _():
        pltpu.sync_copy(stage_v.at[pl.ds(0, 1000)],
                        deg_sh.at[pl.ds(sid * 1000, 1000)])

    plsc.subcore_barrier()

    def body(j, _):
        pltpu.async_copy(ones_v, deg_sh.at[idx_v.at[j]], sem, add=True)
        return ()
    lax.fori_loop(0, NCH, body, ())

    def drain(j, _):
        pltpu.make_async_copy(ones_v, deg_sh.at[idx_v.at[j]], sem).wait()
        return ()
    lax.fori_loop(0, NCH, drain, ())

    plsc.subcore_barrier()

    @pl.when(sid < 10)
    def _():
        pltpu.sync_copy(deg_sh.at[pl.ds(sid * 1000, 1000)],
                        stage_v.at[pl.ds(0, 1000)])
        pltpu.sync_copy(stage_v.at[pl.ds(0, 1000)],
                        out_hbm.at[pl.ds(cid * N + sid * 1000, 1000)])


def _make_agg_kernel(feat):
    @functools.partial(
        pl.kernel,
        out_type=jax.ShapeDtypeStruct((NC, N, feat), jnp.float32),
        mesh=_MESH,
        compiler_params=_SC_PARAMS,
        scratch_types=[
            pltpu.VMEM((NCH, CH), jnp.int32),
            pltpu.VMEM((NCH, CH), jnp.int32),
            [pltpu.VMEM((CH, feat), jnp.float32)] * 4,
            pltpu.VMEM((RPT, feat), jnp.float32),
            pltpu.VMEM_SHARED((N, feat), jnp.float32),
            [pltpu.SemaphoreType.DMA] * 4,
            [pltpu.SemaphoreType.DMA] * 4,
        ],
    )
    def _agg(h_hbm, src_hbm, dst_hbm, out_hbm,
             src_v, dst_v, rows, stage_v, acc_sh, semg, sems):
        cid = lax.axis_index("c")
        sid = lax.axis_index("s")
        wid = cid * NS + sid

        pltpu.sync_copy(src_hbm.at[wid], src_v)
        pltpu.sync_copy(dst_hbm.at[wid], dst_v)
        @pl.when(sid < 15)
        def _():
            pltpu.sync_copy(h_hbm.at[pl.ds(sid * RPT, RPT)], stage_v)
            pltpu.sync_copy(stage_v, acc_sh.at[pl.ds(sid * RPT, RPT)])

        @pl.when(sid == 15)
        def _():
            pltpu.sync_copy(h_hbm.at[pl.ds(15 * RPT, RPT_LAST)],
                            stage_v.at[pl.ds(0, RPT_LAST)])
            pltpu.sync_copy(stage_v.at[pl.ds(0, RPT_LAST)],
                            acc_sh.at[pl.ds(15 * RPT, RPT_LAST)])

        plsc.subcore_barrier()

        def _gather(j, buf, sem):
            pltpu.async_copy(h_hbm.at[src_v.at[j]], buf, sem)

        def _scatter(j, buf, sem):
            pltpu.async_copy(buf, acc_sh.at[dst_v.at[j]], sem, add=True)

        def _wait_gather(j, buf, sem):
            pltpu.make_async_copy(h_hbm.at[src_v.at[j]], buf, sem).wait()

        def _wait_scatter(j, buf, sem):
            pltpu.make_async_copy(buf, acc_sh.at[dst_v.at[j]], sem).wait()

        for b in range(4):
            _gather(b, rows[b], semg[b])

        def group(g, _):
            for b in range(4):
                j = 4 * g + b
                _wait_gather(j, rows[b], semg[b])
                _scatter(j, rows[b], sems[b])

                @pl.when(jnp.logical_and(j >= 2, j <= NCH - 3))
                def _(j=j, b=b):
                    b2 = (b + 2) % 4
                    _wait_scatter(j - 2, rows[b2], sems[b2])
                    _gather(j + 2, rows[b2], semg[b2])
            return ()
        lax.fori_loop(0, NCH // 4, group, ())

        _wait_gather(NCH - 1, rows[0], semg[0])
        _scatter(NCH - 1, rows[0], sems[0])
        for b in range(4):
            _wait_scatter(0, rows[b], sems[b])

        plsc.subcore_barrier()

        @pl.when(sid < 15)
        def _():
            pltpu.sync_copy(acc_sh.at[pl.ds(sid * RPT, RPT)], stage_v)
            pltpu.sync_copy(stage_v,
                            out_hbm.at[cid, pl.ds(sid * RPT, RPT)])

        @pl.when(sid == 15)
        def _():
            pltpu.sync_copy(acc_sh.at[pl.ds(15 * RPT, RPT_LAST)],
                            stage_v.at[pl.ds(0, RPT_LAST)])
            pltpu.sync_copy(stage_v.at[pl.ds(0, RPT_LAST)],
                            out_hbm.at[cid, pl.ds(15 * RPT, RPT_LAST)])

    return _agg


_agg16 = _make_agg_kernel(H)
_agg40 = _make_agg_kernel(C)


_RB = 1000
_GRID = N // _RB


def _tc_mm_body(x_ref, w1_ref, h1_ref):
    h1_ref[...] = jnp.dot(x_ref[...], w1_ref[...],
                          preferred_element_type=jnp.float32)


def _tc_mm(x, w1):
    return pl.pallas_call(
        _tc_mm_body,
        grid=(_GRID,),
        in_specs=[
            pl.BlockSpec((_RB, D_IN), lambda i: (i, 0)),
            pl.BlockSpec((D_IN, H), lambda i: (0, 0)),
        ],
        out_specs=pl.BlockSpec((_RB, H), lambda i: (i, 0)),
        out_shape=jax.ShapeDtypeStruct((N, H), jnp.float32),
    )(x, w1)


def _tc1_body(h1_ref, degp_ref, h1p_ref, dis_ref):
    deg = degp_ref[0] + degp_ref[1] + 1.0
    dis = lax.rsqrt(deg)
    h1p_ref[...] = h1_ref[...] * dis
    dis_ref[...] = dis


def _tc1(h1, degp):
    return pl.pallas_call(
        _tc1_body,
        grid=(_GRID,),
        in_specs=[
            pl.BlockSpec((_RB, H), lambda i: (i, 0)),
            pl.BlockSpec((NC, _RB, 1), lambda i: (0, i, 0)),
        ],
        out_specs=[
            pl.BlockSpec((_RB, H), lambda i: (i, 0)),
            pl.BlockSpec((_RB, 1), lambda i: (i, 0)),
        ],
        out_shape=[
            jax.ShapeDtypeStruct((N, H), jnp.float32),
            jax.ShapeDtypeStruct((N, 1), jnp.float32),
        ],
    )(h1, degp)


def _tc2_body(p_ref, h1p_ref, dis_ref, b1_ref, w2_ref, h2p_ref):
    dis = dis_ref[...]
    agg = p_ref[0] + p_ref[1] - h1p_ref[...]
    a1 = jnp.maximum(agg * dis + b1_ref[...], 0.0)
    h2 = jnp.dot(a1, w2_ref[...], preferred_element_type=jnp.float32)
    h2p_ref[...] = h2 * dis


def _tc2(p, h1p, dis, b1, w2):
    return pl.pallas_call(
        _tc2_body,
        grid=(_GRID,),
        in_specs=[
            pl.BlockSpec((NC, _RB, H), lambda i: (0, i, 0)),
            pl.BlockSpec((_RB, H), lambda i: (i, 0)),
            pl.BlockSpec((_RB, 1), lambda i: (i, 0)),
            pl.BlockSpec((1, H), lambda i: (0, 0)),
            pl.BlockSpec((H, C), lambda i: (0, 0)),
        ],
        out_specs=pl.BlockSpec((_RB, C), lambda i: (i, 0)),
        out_shape=jax.ShapeDtypeStruct((N, C), jnp.float32),
    )(p, h1p, dis, b1, w2)


def _tc3_body(q_ref, h2p_ref, dis_ref, b2_ref, out_ref):
    z = (q_ref[0] + q_ref[1] - h2p_ref[...]) * dis_ref[...] + b2_ref[...]
    m = jnp.max(z, axis=1, keepdims=True)
    s = jnp.sum(jnp.exp(z - m), axis=1, keepdims=True)
    out_ref[...] = z - m - jnp.log(s)


def _tc3(q, h2p, dis, b2):
    return pl.pallas_call(
        _tc3_body,
        grid=(_GRID,),
        in_specs=[
            pl.BlockSpec((NC, _RB, C), lambda i: (0, i, 0)),
            pl.BlockSpec((_RB, C), lambda i: (i, 0)),
            pl.BlockSpec((_RB, 1), lambda i: (i, 0)),
            pl.BlockSpec((1, C), lambda i: (0, 0)),
        ],
        out_specs=pl.BlockSpec((_RB, C), lambda i: (i, 0)),
        out_shape=jax.ShapeDtypeStruct((N, C), jnp.float32),
    )(q, h2p, dis, b2)


@jax.jit
def kernel(x, edge_index, W1, b1, W2, b2):
    src = edge_index[0].reshape(NW, NCH, CH)
    dst = edge_index[1].reshape(NW, NCH, CH)

    degp = _deg_kernel(dst)
    h1 = _tc_mm(x, W1)
    h1p, dis = _tc1(h1, degp.reshape(NC, N, 1))
    p = _agg16(h1p, src, dst)
    h2p = _tc2(p, h1p, dis, b1.reshape(1, H), W2)
    q = _agg40(h2p, src, dst)
    return _tc3(q, h2p, dis, b2.reshape(1, C))


# speedup vs baseline: 44.7565x; 1.1034x over previous
import functools

import jax
import jax.numpy as jnp
from jax import lax
from jax.experimental import pallas as pl
from jax.experimental.pallas import tpu as pltpu
from jax.experimental.pallas import tpu_sc as plsc

N = 10000
E = 320000
D_IN = 128
H = 16
C = 40

NC = 2
NS = 16
NW = NC * NS
EPT = E // NW
CH = 80
NCH = EPT // CH
RPT = 632
RPT_LAST = N - 15 * RPT

_MESH = plsc.VectorSubcoreMesh(core_axis_name="c", subcore_axis_name="s")
_SC_PARAMS = pltpu.CompilerParams(use_tc_tiling_on_sc=False)


@functools.partial(
    pl.kernel,
    out_type=jax.ShapeDtypeStruct((NC * N,), jnp.float32),
    mesh=_MESH,
    compiler_params=_SC_PARAMS,
    scratch_types=[
        pltpu.VMEM((NCH, CH), jnp.int32),
        pltpu.VMEM((CH,), jnp.float32),
        pltpu.VMEM((1008,), jnp.float32),
        pltpu.VMEM_SHARED((N,), jnp.float32),
        pltpu.SemaphoreType.DMA,
        pltpu.SemaphoreType.DMA,
    ],
)
def _deg_kernel(edge_hbm, out_hbm, idx_v, ones_v, stage_v, deg_sh, sem, semi):
    cid = lax.axis_index("c")
    sid = lax.axis_index("s")
    wid = cid * NS + sid

    def fill_idx(j, _):
        pltpu.async_copy(edge_hbm.at[1, pl.ds(wid * EPT + j * CH, CH)],
                         idx_v.at[j], semi)
        return ()
    lax.fori_loop(0, NCH, fill_idx, ())

    def drain_idx(j, _):
        pltpu.make_async_copy(edge_hbm.at[1, pl.ds(wid * EPT + j * CH, CH)],
                              idx_v.at[j], semi).wait()
        return ()
    lax.fori_loop(0, NCH, drain_idx, ())

    def fill_ones(i, _):
        ones_v[pl.ds(i * 16, 16)] = jnp.ones((16,), jnp.float32)
        return ()
    lax.fori_loop(0, CH // 16, fill_ones, ())

    def fill_zeros(i, _):
        stage_v[pl.ds(i * 16, 16)] = jnp.zeros((16,), jnp.float32)
        return ()
    lax.fori_loop(0, 63, fill_zeros, ())

    @pl.when(sid < 10)
    def _():
        pltpu.sync_copy(stage_v.at[pl.ds(0, 1000)],
                        deg_sh.at[pl.ds(sid * 1000, 1000)])

    plsc.subcore_barrier()

    def body(j, _):
        pltpu.async_copy(ones_v, deg_sh.at[idx_v.at[j]], sem, add=True)
        return ()
    lax.fori_loop(0, NCH, body, ())

    def drain(j, _):
        pltpu.make_async_copy(ones_v, deg_sh.at[idx_v.at[j]], sem).wait()
        return ()
    lax.fori_loop(0, NCH, drain, ())

    plsc.subcore_barrier()

    @pl.when(sid < 10)
    def _():
        pltpu.sync_copy(deg_sh.at[pl.ds(sid * 1000, 1000)],
                        stage_v.at[pl.ds(0, 1000)])
        pltpu.sync_copy(stage_v.at[pl.ds(0, 1000)],
                        out_hbm.at[pl.ds(cid * N + sid * 1000, 1000)])


def _make_agg_kernel(feat):
    @functools.partial(
        pl.kernel,
        out_type=jax.ShapeDtypeStruct((NC, N, feat), jnp.float32),
        mesh=_MESH,
        compiler_params=_SC_PARAMS,
        scratch_types=[
            pltpu.VMEM((NCH, CH), jnp.int32),
            pltpu.VMEM((NCH, CH), jnp.int32),
            [pltpu.VMEM((CH, feat), jnp.float32)] * 4,
            pltpu.VMEM((RPT, feat), jnp.float32),
            pltpu.VMEM_SHARED((N, feat), jnp.float32),
            [pltpu.SemaphoreType.DMA] * 4,
            [pltpu.SemaphoreType.DMA] * 4,
            pltpu.SemaphoreType.DMA,
        ],
    )
    def _agg(h_hbm, edge_hbm, out_hbm,
             src_v, dst_v, rows, stage_v, acc_sh, semg, sems, semi):
        cid = lax.axis_index("c")
        sid = lax.axis_index("s")
        wid = cid * NS + sid

        def fill_idx(j, _):
            base = wid * EPT + j * CH
            pltpu.async_copy(edge_hbm.at[0, pl.ds(base, CH)],
                             src_v.at[j], semi)
            pltpu.async_copy(edge_hbm.at[1, pl.ds(base, CH)],
                             dst_v.at[j], semi)
            return ()
        lax.fori_loop(0, NCH, fill_idx, ())

        def drain_idx(j, _):
            base = wid * EPT + j * CH
            pltpu.make_async_copy(edge_hbm.at[0, pl.ds(base, CH)],
                                  src_v.at[j], semi).wait()
            pltpu.make_async_copy(edge_hbm.at[1, pl.ds(base, CH)],
                                  dst_v.at[j], semi).wait()
            return ()
        lax.fori_loop(0, NCH, drain_idx, ())
        @pl.when(sid < 15)
        def _():
            pltpu.sync_copy(h_hbm.at[pl.ds(sid * RPT, RPT)], stage_v)
            pltpu.sync_copy(stage_v, acc_sh.at[pl.ds(sid * RPT, RPT)])

        @pl.when(sid == 15)
        def _():
            pltpu.sync_copy(h_hbm.at[pl.ds(15 * RPT, RPT_LAST)],
                            stage_v.at[pl.ds(0, RPT_LAST)])
            pltpu.sync_copy(stage_v.at[pl.ds(0, RPT_LAST)],
                            acc_sh.at[pl.ds(15 * RPT, RPT_LAST)])

        plsc.subcore_barrier()

        def _gather(j, buf, sem):
            pltpu.async_copy(h_hbm.at[src_v.at[j]], buf, sem)

        def _scatter(j, buf, sem):
            pltpu.async_copy(buf, acc_sh.at[dst_v.at[j]], sem, add=True)

        def _wait_gather(j, buf, sem):
            pltpu.make_async_copy(h_hbm.at[src_v.at[j]], buf, sem).wait()

        def _wait_scatter(j, buf, sem):
            pltpu.make_async_copy(buf, acc_sh.at[dst_v.at[j]], sem).wait()

        for b in range(4):
            _gather(b, rows[b], semg[b])

        def group(g, _):
            for b in range(4):
                j = 4 * g + b
                _wait_gather(j, rows[b], semg[b])
                _scatter(j, rows[b], sems[b])

                @pl.when(jnp.logical_and(j >= 2, j <= NCH - 3))
                def _(j=j, b=b):
                    b2 = (b + 2) % 4
                    _wait_scatter(j - 2, rows[b2], sems[b2])
                    _gather(j + 2, rows[b2], semg[b2])
            return ()
        lax.fori_loop(0, NCH // 4, group, ())

        _wait_gather(NCH - 1, rows[0], semg[0])
        _scatter(NCH - 1, rows[0], sems[0])
        for b in range(4):
            _wait_scatter(0, rows[b], sems[b])

        plsc.subcore_barrier()

        @pl.when(sid < 15)
        def _():
            pltpu.sync_copy(acc_sh.at[pl.ds(sid * RPT, RPT)], stage_v)
            pltpu.sync_copy(stage_v,
                            out_hbm.at[cid, pl.ds(sid * RPT, RPT)])

        @pl.when(sid == 15)
        def _():
            pltpu.sync_copy(acc_sh.at[pl.ds(15 * RPT, RPT_LAST)],
                            stage_v.at[pl.ds(0, RPT_LAST)])
            pltpu.sync_copy(stage_v.at[pl.ds(0, RPT_LAST)],
                            out_hbm.at[cid, pl.ds(15 * RPT, RPT_LAST)])

    return _agg


_agg16 = _make_agg_kernel(H)
_agg40 = _make_agg_kernel(C)


_RB = 2000
_GRID = N // _RB


def _tc1_body(x_ref, w1_ref, degp_ref, h1p_ref, dis_ref):
    deg = degp_ref[0] + degp_ref[1] + 1.0
    dis = lax.rsqrt(deg)
    h = jnp.dot(x_ref[...], w1_ref[...], preferred_element_type=jnp.float32)
    h1p_ref[...] = h * dis
    dis_ref[...] = dis


def _tc1(x, w1, degp):
    return pl.pallas_call(
        _tc1_body,
        grid=(_GRID,),
        in_specs=[
            pl.BlockSpec((_RB, D_IN), lambda i: (i, 0)),
            pl.BlockSpec((D_IN, H), lambda i: (0, 0)),
            pl.BlockSpec((NC, _RB, 1), lambda i: (0, i, 0)),
        ],
        out_specs=[
            pl.BlockSpec((_RB, H), lambda i: (i, 0)),
            pl.BlockSpec((_RB, 1), lambda i: (i, 0)),
        ],
        out_shape=[
            jax.ShapeDtypeStruct((N, H), jnp.float32),
            jax.ShapeDtypeStruct((N, 1), jnp.float32),
        ],
    )(x, w1, degp)


def _tc2_body(p_ref, h1p_ref, dis_ref, b1_ref, w2_ref, h2p_ref):
    dis = dis_ref[...]
    agg = p_ref[0] + p_ref[1] - h1p_ref[...]
    a1 = jnp.maximum(agg * dis + b1_ref[...], 0.0)
    h2 = jnp.dot(a1, w2_ref[...], preferred_element_type=jnp.float32)
    h2p_ref[...] = h2 * dis


def _tc2(p, h1p, dis, b1, w2):
    return pl.pallas_call(
        _tc2_body,
        grid=(_GRID,),
        in_specs=[
            pl.BlockSpec((NC, _RB, H), lambda i: (0, i, 0)),
            pl.BlockSpec((_RB, H), lambda i: (i, 0)),
            pl.BlockSpec((_RB, 1), lambda i: (i, 0)),
            pl.BlockSpec((1, H), lambda i: (0, 0)),
            pl.BlockSpec((H, C), lambda i: (0, 0)),
        ],
        out_specs=pl.BlockSpec((_RB, C), lambda i: (i, 0)),
        out_shape=jax.ShapeDtypeStruct((N, C), jnp.float32),
    )(p, h1p, dis, b1, w2)


def _tc3_body(q_ref, h2p_ref, dis_ref, b2_ref, out_ref):
    z = (q_ref[0] + q_ref[1] - h2p_ref[...]) * dis_ref[...] + b2_ref[...]
    m = jnp.max(z, axis=1, keepdims=True)
    s = jnp.sum(jnp.exp(z - m), axis=1, keepdims=True)
    out_ref[...] = z - m - jnp.log(s)


def _tc3(q, h2p, dis, b2):
    return pl.pallas_call(
        _tc3_body,
        grid=(_GRID,),
        in_specs=[
            pl.BlockSpec((NC, _RB, C), lambda i: (0, i, 0)),
            pl.BlockSpec((_RB, C), lambda i: (i, 0)),
            pl.BlockSpec((_RB, 1), lambda i: (i, 0)),
            pl.BlockSpec((1, C), lambda i: (0, 0)),
        ],
        out_specs=pl.BlockSpec((_RB, C), lambda i: (i, 0)),
        out_shape=jax.ShapeDtypeStruct((N, C), jnp.float32),
    )(q, h2p, dis, b2)


@jax.jit
def kernel(x, edge_index, W1, b1, W2, b2):
    degp = _deg_kernel(edge_index)
    h1p, dis = _tc1(x, W1, degp.reshape(NC, N, 1))
    p = _agg16(h1p, edge_index)
    h2p = _tc2(p, h1p, dis, b1.reshape(1, H), W2)
    q = _agg40(h2p, edge_index)
    return _tc3(q, h2p, dis, b2.reshape(1, C))


# speedup vs baseline: 51.8392x; 1.1583x over previous
import functools

import jax
import jax.numpy as jnp
from jax import lax
from jax.experimental import pallas as pl
from jax.experimental.pallas import tpu as pltpu
from jax.experimental.pallas import tpu_sc as plsc

N = 10000
E = 320000
D_IN = 128
H = 16
C = 40

NC = 2
NS = 16
NW = NC * NS
EPT = E // NW
CH = 80
NCH = EPT // CH
RPT = 632
RPT_LAST = N - 15 * RPT

_MESH = plsc.VectorSubcoreMesh(core_axis_name="c", subcore_axis_name="s")
_SC_PARAMS = pltpu.CompilerParams(use_tc_tiling_on_sc=False)


@functools.partial(
    pl.kernel,
    out_type=jax.ShapeDtypeStruct((NC * N,), jnp.float32),
    mesh=_MESH,
    compiler_params=_SC_PARAMS,
    scratch_types=[
        pltpu.VMEM((NCH, CH), jnp.int32),
        pltpu.VMEM((CH,), jnp.float32),
        pltpu.VMEM((1008,), jnp.float32),
        pltpu.VMEM_SHARED((N,), jnp.float32),
        pltpu.SemaphoreType.DMA,
        pltpu.SemaphoreType.DMA,
    ],
)
def _deg_kernel(edge_hbm, out_hbm, idx_v, ones_v, stage_v, deg_sh, sem, semi):
    cid = lax.axis_index("c")
    sid = lax.axis_index("s")
    wid = cid * NS + sid

    def fill_idx(j, _):
        pltpu.async_copy(edge_hbm.at[1, pl.ds(wid * EPT + j * CH, CH)],
                         idx_v.at[j], semi)
        return ()
    lax.fori_loop(0, NCH, fill_idx, ())

    def drain_idx(j, _):
        pltpu.make_async_copy(edge_hbm.at[1, pl.ds(wid * EPT + j * CH, CH)],
                              idx_v.at[j], semi).wait()
        return ()
    lax.fori_loop(0, NCH, drain_idx, ())

    def fill_ones(i, _):
        ones_v[pl.ds(i * 16, 16)] = jnp.ones((16,), jnp.float32)
        return ()
    lax.fori_loop(0, CH // 16, fill_ones, ())

    def fill_zeros(i, _):
        stage_v[pl.ds(i * 16, 16)] = jnp.zeros((16,), jnp.float32)
        return ()
    lax.fori_loop(0, 63, fill_zeros, ())

    @pl.when(sid < 10)
    def _():
        pltpu.sync_copy(stage_v.at[pl.ds(0, 1000)],
                        deg_sh.at[pl.ds(sid * 1000, 1000)])

    plsc.subcore_barrier()

    def body(j, _):
        pltpu.async_copy(ones_v, deg_sh.at[idx_v.at[j]], sem, add=True)
        return ()
    lax.fori_loop(0, NCH, body, ())

    def drain(j, _):
        pltpu.make_async_copy(ones_v, deg_sh.at[idx_v.at[j]], sem).wait()
        return ()
    lax.fori_loop(0, NCH, drain, ())

    plsc.subcore_barrier()

    @pl.when(sid < 10)
    def _():
        pltpu.sync_copy(deg_sh.at[pl.ds(sid * 1000, 1000)],
                        stage_v.at[pl.ds(0, 1000)])
        pltpu.sync_copy(stage_v.at[pl.ds(0, 1000)],
                        out_hbm.at[pl.ds(cid * N + sid * 1000, 1000)])


def _make_agg_kernel(feat, table_in_spmem=False):
    extra = ([pltpu.VMEM_SHARED((N, feat), jnp.float32)]
             if table_in_spmem else [])
    @functools.partial(
        pl.kernel,
        out_type=jax.ShapeDtypeStruct((NC, N, feat), jnp.float32),
        mesh=_MESH,
        compiler_params=_SC_PARAMS,
        scratch_types=[
            pltpu.VMEM((NCH, CH), jnp.int32),
            pltpu.VMEM((NCH, CH), jnp.int32),
            [pltpu.VMEM((CH, feat), jnp.float32)] * 4,
            pltpu.VMEM((RPT, feat), jnp.float32),
            pltpu.VMEM_SHARED((N, feat), jnp.float32),
            [pltpu.SemaphoreType.DMA] * 4,
            [pltpu.SemaphoreType.DMA] * 4,
            pltpu.SemaphoreType.DMA,
        ] + extra,
    )
    def _agg(h_hbm, edge_hbm, out_hbm,
             src_v, dst_v, rows, stage_v, acc_sh, semg, sems, semi,
             *maybe_table):
        table = maybe_table[0] if maybe_table else h_hbm
        cid = lax.axis_index("c")
        sid = lax.axis_index("s")
        wid = cid * NS + sid

        def fill_idx(j, _):
            base = wid * EPT + j * CH
            pltpu.async_copy(edge_hbm.at[0, pl.ds(base, CH)],
                             src_v.at[j], semi)
            pltpu.async_copy(edge_hbm.at[1, pl.ds(base, CH)],
                             dst_v.at[j], semi)
            return ()
        lax.fori_loop(0, NCH, fill_idx, ())

        def drain_idx(j, _):
            base = wid * EPT + j * CH
            pltpu.make_async_copy(edge_hbm.at[0, pl.ds(base, CH)],
                                  src_v.at[j], semi).wait()
            pltpu.make_async_copy(edge_hbm.at[1, pl.ds(base, CH)],
                                  dst_v.at[j], semi).wait()
            return ()
        lax.fori_loop(0, NCH, drain_idx, ())
        @pl.when(sid < 15)
        def _():
            pltpu.sync_copy(h_hbm.at[pl.ds(sid * RPT, RPT)], stage_v)
            pltpu.sync_copy(stage_v, acc_sh.at[pl.ds(sid * RPT, RPT)])
            if table_in_spmem:
                pltpu.sync_copy(stage_v, table.at[pl.ds(sid * RPT, RPT)])

        @pl.when(sid == 15)
        def _():
            pltpu.sync_copy(h_hbm.at[pl.ds(15 * RPT, RPT_LAST)],
                            stage_v.at[pl.ds(0, RPT_LAST)])
            pltpu.sync_copy(stage_v.at[pl.ds(0, RPT_LAST)],
                            acc_sh.at[pl.ds(15 * RPT, RPT_LAST)])
            if table_in_spmem:
                pltpu.sync_copy(stage_v.at[pl.ds(0, RPT_LAST)],
                                table.at[pl.ds(15 * RPT, RPT_LAST)])

        plsc.subcore_barrier()

        def _gather(j, buf, sem):
            pltpu.async_copy(table.at[src_v.at[j]], buf, sem)

        def _scatter(j, buf, sem):
            pltpu.async_copy(buf, acc_sh.at[dst_v.at[j]], sem, add=True)

        def _wait_gather(j, buf, sem):
            pltpu.make_async_copy(table.at[src_v.at[j]], buf, sem).wait()

        def _wait_scatter(j, buf, sem):
            pltpu.make_async_copy(buf, acc_sh.at[dst_v.at[j]], sem).wait()

        for b in range(4):
            _gather(b, rows[b], semg[b])

        def group(g, _):
            for b in range(4):
                j = 4 * g + b
                _wait_gather(j, rows[b], semg[b])
                _scatter(j, rows[b], sems[b])

                @pl.when(jnp.logical_and(j >= 2, j <= NCH - 3))
                def _(j=j, b=b):
                    b2 = (b + 2) % 4
                    _wait_scatter(j - 2, rows[b2], sems[b2])
                    _gather(j + 2, rows[b2], semg[b2])
            return ()
        lax.fori_loop(0, NCH // 4, group, ())

        _wait_gather(NCH - 1, rows[0], semg[0])
        _scatter(NCH - 1, rows[0], sems[0])
        for b in range(4):
            _wait_scatter(0, rows[b], sems[b])

        plsc.subcore_barrier()

        @pl.when(sid < 15)
        def _():
            pltpu.sync_copy(acc_sh.at[pl.ds(sid * RPT, RPT)], stage_v)
            pltpu.sync_copy(stage_v,
                            out_hbm.at[cid, pl.ds(sid * RPT, RPT)])

        @pl.when(sid == 15)
        def _():
            pltpu.sync_copy(acc_sh.at[pl.ds(15 * RPT, RPT_LAST)],
                            stage_v.at[pl.ds(0, RPT_LAST)])
            pltpu.sync_copy(stage_v.at[pl.ds(0, RPT_LAST)],
                            out_hbm.at[cid, pl.ds(15 * RPT, RPT_LAST)])

    return _agg


_agg16 = _make_agg_kernel(H, table_in_spmem=True)
_agg40 = _make_agg_kernel(C)


_RB = 2000
_GRID = N // _RB


def _tc1_body(x_ref, w1_ref, degp_ref, h1p_ref, dis_ref):
    deg = degp_ref[0] + degp_ref[1] + 1.0
    dis = lax.rsqrt(deg)
    h = jnp.dot(x_ref[...], w1_ref[...], preferred_element_type=jnp.float32)
    h1p_ref[...] = h * dis
    dis_ref[...] = dis


def _tc1(x, w1, degp):
    return pl.pallas_call(
        _tc1_body,
        grid=(_GRID,),
        in_specs=[
            pl.BlockSpec((_RB, D_IN), lambda i: (i, 0)),
            pl.BlockSpec((D_IN, H), lambda i: (0, 0)),
            pl.BlockSpec((NC, _RB, 1), lambda i: (0, i, 0)),
        ],
        out_specs=[
            pl.BlockSpec((_RB, H), lambda i: (i, 0)),
            pl.BlockSpec((_RB, 1), lambda i: (i, 0)),
        ],
        out_shape=[
            jax.ShapeDtypeStruct((N, H), jnp.float32),
            jax.ShapeDtypeStruct((N, 1), jnp.float32),
        ],
    )(x, w1, degp)


def _tc2_body(p_ref, h1p_ref, dis_ref, b1_ref, w2_ref, h2p_ref):
    dis = dis_ref[...]
    agg = p_ref[0] + p_ref[1] - h1p_ref[...]
    a1 = jnp.maximum(agg * dis + b1_ref[...], 0.0)
    h2 = jnp.dot(a1, w2_ref[...], preferred_element_type=jnp.float32)
    h2p_ref[...] = h2 * dis


def _tc2(p, h1p, dis, b1, w2):
    return pl.pallas_call(
        _tc2_body,
        grid=(_GRID,),
        in_specs=[
            pl.BlockSpec((NC, _RB, H), lambda i: (0, i, 0)),
            pl.BlockSpec((_RB, H), lambda i: (i, 0)),
            pl.BlockSpec((_RB, 1), lambda i: (i, 0)),
            pl.BlockSpec((1, H), lambda i: (0, 0)),
            pl.BlockSpec((H, C), lambda i: (0, 0)),
        ],
        out_specs=pl.BlockSpec((_RB, C), lambda i: (i, 0)),
        out_shape=jax.ShapeDtypeStruct((N, C), jnp.float32),
    )(p, h1p, dis, b1, w2)


def _tc3_body(q_ref, h2p_ref, dis_ref, b2_ref, out_ref):
    z = (q_ref[0] + q_ref[1] - h2p_ref[...]) * dis_ref[...] + b2_ref[...]
    m = jnp.max(z, axis=1, keepdims=True)
    s = jnp.sum(jnp.exp(z - m), axis=1, keepdims=True)
    out_ref[...] = z - m - jnp.log(s)


def _tc3(q, h2p, dis, b2):
    return pl.pallas_call(
        _tc3_body,
        grid=(_GRID,),
        in_specs=[
            pl.BlockSpec((NC, _RB, C), lambda i: (0, i, 0)),
            pl.BlockSpec((_RB, C), lambda i: (i, 0)),
            pl.BlockSpec((_RB, 1), lambda i: (i, 0)),
            pl.BlockSpec((1, C), lambda i: (0, 0)),
        ],
        out_specs=pl.BlockSpec((_RB, C), lambda i: (i, 0)),
        out_shape=jax.ShapeDtypeStruct((N, C), jnp.float32),
    )(q, h2p, dis, b2)


@jax.jit
def kernel(x, edge_index, W1, b1, W2, b2):
    degp = _deg_kernel(edge_index)
    h1p, dis = _tc1(x, W1, degp.reshape(NC, N, 1))
    p = _agg16(h1p, edge_index)
    h2p = _tc2(p, h1p, dis, b1.reshape(1, H), W2)
    q = _agg40(h2p, edge_index)
    return _tc3(q, h2p, dis, b2.reshape(1, C))


# speedup vs baseline: 57.3154x; 1.1056x over previous
import functools

import jax
import jax.numpy as jnp
from jax import lax
from jax.experimental import pallas as pl
from jax.experimental.pallas import tpu as pltpu
from jax.experimental.pallas import tpu_sc as plsc

N = 10000
E = 320000
D_IN = 128
H = 16
C = 40

NC = 2
NS = 16
NW = NC * NS
EPT = E // NW
CH = 80
NCH = EPT // CH
RPT = 632
RPT_LAST = N - 15 * RPT

_MESH = plsc.VectorSubcoreMesh(core_axis_name="c", subcore_axis_name="s")
_SC_PARAMS = pltpu.CompilerParams(use_tc_tiling_on_sc=False)


@functools.partial(
    pl.kernel,
    out_type=jax.ShapeDtypeStruct((NC * N,), jnp.float32),
    mesh=_MESH,
    compiler_params=_SC_PARAMS,
    scratch_types=[
        pltpu.VMEM((NCH, CH), jnp.int32),
        pltpu.VMEM((CH,), jnp.float32),
        pltpu.VMEM((1008,), jnp.float32),
        pltpu.VMEM_SHARED((N,), jnp.float32),
        pltpu.SemaphoreType.DMA,
        pltpu.SemaphoreType.DMA,
    ],
)
def _deg_kernel(edge_hbm, out_hbm, idx_v, ones_v, stage_v, deg_sh, sem, semi):
    cid = lax.axis_index("c")
    sid = lax.axis_index("s")
    wid = cid * NS + sid

    def fill_idx(j, _):
        pltpu.async_copy(edge_hbm.at[1, pl.ds(wid * EPT + j * CH, CH)],
                         idx_v.at[j], semi)
        return ()
    lax.fori_loop(0, NCH, fill_idx, ())

    def drain_idx(j, _):
        pltpu.make_async_copy(edge_hbm.at[1, pl.ds(wid * EPT + j * CH, CH)],
                              idx_v.at[j], semi).wait()
        return ()
    lax.fori_loop(0, NCH, drain_idx, ())

    def fill_ones(i, _):
        ones_v[pl.ds(i * 16, 16)] = jnp.ones((16,), jnp.float32)
        return ()
    lax.fori_loop(0, CH // 16, fill_ones, ())

    def fill_zeros(i, _):
        stage_v[pl.ds(i * 16, 16)] = jnp.zeros((16,), jnp.float32)
        return ()
    lax.fori_loop(0, 63, fill_zeros, ())

    @pl.when(sid < 10)
    def _():
        pltpu.sync_copy(stage_v.at[pl.ds(0, 1000)],
                        deg_sh.at[pl.ds(sid * 1000, 1000)])

    plsc.subcore_barrier()

    def body(j, _):
        pltpu.async_copy(ones_v, deg_sh.at[idx_v.at[j]], sem, add=True)
        return ()
    lax.fori_loop(0, NCH, body, ())

    def drain(j, _):
        pltpu.make_async_copy(ones_v, deg_sh.at[idx_v.at[j]], sem).wait()
        return ()
    lax.fori_loop(0, NCH, drain, ())

    plsc.subcore_barrier()

    @pl.when(sid < 10)
    def _():
        pltpu.sync_copy(deg_sh.at[pl.ds(sid * 1000, 1000)],
                        stage_v.at[pl.ds(0, 1000)])
        pltpu.sync_copy(stage_v.at[pl.ds(0, 1000)],
                        out_hbm.at[pl.ds(cid * N + sid * 1000, 1000)])


def _make_agg_kernel(feat, table_in_spmem=False):
    extra = ([pltpu.VMEM_SHARED((N, feat), jnp.float32)]
             if table_in_spmem else [])
    @functools.partial(
        pl.kernel,
        out_type=jax.ShapeDtypeStruct((NC, N, feat), jnp.float32),
        mesh=_MESH,
        compiler_params=_SC_PARAMS,
        scratch_types=[
            pltpu.VMEM((NCH, CH), jnp.int32),
            pltpu.VMEM((NCH, CH), jnp.int32),
            [pltpu.VMEM((CH, feat), jnp.float32)] * 4,
            pltpu.VMEM((RPT, feat), jnp.float32),
            pltpu.VMEM_SHARED((N, feat), jnp.float32),
            [pltpu.SemaphoreType.DMA] * 4,
            [pltpu.SemaphoreType.DMA] * 4,
            pltpu.SemaphoreType.DMA,
        ] + extra,
    )
    def _agg(h_hbm, edge_hbm, out_hbm,
             src_v, dst_v, rows, stage_v, acc_sh, semg, sems, semi,
             *maybe_table):
        table = maybe_table[0] if maybe_table else h_hbm
        cid = lax.axis_index("c")
        sid = lax.axis_index("s")
        wid = cid * NS + sid

        def fill_idx(j, _):
            base = wid * EPT + j * CH
            pltpu.async_copy(edge_hbm.at[0, pl.ds(base, CH)],
                             src_v.at[j], semi)
            pltpu.async_copy(edge_hbm.at[1, pl.ds(base, CH)],
                             dst_v.at[j], semi)
            return ()
        lax.fori_loop(0, NCH, fill_idx, ())

        def drain_idx(j, _):
            base = wid * EPT + j * CH
            pltpu.make_async_copy(edge_hbm.at[0, pl.ds(base, CH)],
                                  src_v.at[j], semi).wait()
            pltpu.make_async_copy(edge_hbm.at[1, pl.ds(base, CH)],
                                  dst_v.at[j], semi).wait()
            return ()
        lax.fori_loop(0, NCH, drain_idx, ())
        @pl.when(sid < 15)
        def _():
            pltpu.sync_copy(h_hbm.at[pl.ds(sid * RPT, RPT)], stage_v)
            pltpu.sync_copy(stage_v, acc_sh.at[pl.ds(sid * RPT, RPT)])
            if table_in_spmem:
                pltpu.sync_copy(stage_v, table.at[pl.ds(sid * RPT, RPT)])

        @pl.when(sid == 15)
        def _():
            pltpu.sync_copy(h_hbm.at[pl.ds(15 * RPT, RPT_LAST)],
                            stage_v.at[pl.ds(0, RPT_LAST)])
            pltpu.sync_copy(stage_v.at[pl.ds(0, RPT_LAST)],
                            acc_sh.at[pl.ds(15 * RPT, RPT_LAST)])
            if table_in_spmem:
                pltpu.sync_copy(stage_v.at[pl.ds(0, RPT_LAST)],
                                table.at[pl.ds(15 * RPT, RPT_LAST)])

        plsc.subcore_barrier()

        def _gather(j, buf, sem):
            pltpu.async_copy(table.at[src_v.at[j]], buf, sem)

        def _scatter(j, buf, sem):
            pltpu.async_copy(buf, acc_sh.at[dst_v.at[j]], sem, add=True)

        def _wait_gather(j, buf, sem):
            pltpu.make_async_copy(table.at[src_v.at[j]], buf, sem).wait()

        def _wait_scatter(j, buf, sem):
            pltpu.make_async_copy(buf, acc_sh.at[dst_v.at[j]], sem).wait()

        for b in range(4):
            _gather(b, rows[b], semg[b])

        def group(g, _):
            for b in range(4):
                j = 4 * g + b
                _wait_gather(j, rows[b], semg[b])
                _scatter(j, rows[b], sems[b])

                @pl.when(jnp.logical_and(j >= 2, j <= NCH - 3))
                def _(j=j, b=b):
                    b2 = (b + 2) % 4
                    _wait_scatter(j - 2, rows[b2], sems[b2])
                    _gather(j + 2, rows[b2], semg[b2])
            return ()
        lax.fori_loop(0, NCH // 4, group, ())

        _wait_gather(NCH - 1, rows[0], semg[0])
        _scatter(NCH - 1, rows[0], sems[0])
        for b in range(4):
            _wait_scatter(0, rows[b], sems[b])

        plsc.subcore_barrier()

        @pl.when(sid < 15)
        def _():
            pltpu.sync_copy(acc_sh.at[pl.ds(sid * RPT, RPT)], stage_v)
            pltpu.sync_copy(stage_v,
                            out_hbm.at[cid, pl.ds(sid * RPT, RPT)])

        @pl.when(sid == 15)
        def _():
            pltpu.sync_copy(acc_sh.at[pl.ds(15 * RPT, RPT_LAST)],
                            stage_v.at[pl.ds(0, RPT_LAST)])
            pltpu.sync_copy(stage_v.at[pl.ds(0, RPT_LAST)],
                            out_hbm.at[cid, pl.ds(15 * RPT, RPT_LAST)])

    return _agg


_agg16 = _make_agg_kernel(H, table_in_spmem=True)
_agg40 = _make_agg_kernel(C, table_in_spmem=True)


_RB = 2000
_GRID = N // _RB


def _tc1_body(x_ref, w1_ref, degp_ref, h1p_ref, dis_ref):
    deg = degp_ref[0] + degp_ref[1] + 1.0
    dis = lax.rsqrt(deg)
    h = jnp.dot(x_ref[...], w1_ref[...], preferred_element_type=jnp.float32)
    h1p_ref[...] = h * dis
    dis_ref[...] = dis


def _tc1(x, w1, degp):
    return pl.pallas_call(
        _tc1_body,
        grid=(_GRID,),
        in_specs=[
            pl.BlockSpec((_RB, D_IN), lambda i: (i, 0)),
            pl.BlockSpec((D_IN, H), lambda i: (0, 0)),
            pl.BlockSpec((NC, _RB, 1), lambda i: (0, i, 0)),
        ],
        out_specs=[
            pl.BlockSpec((_RB, H), lambda i: (i, 0)),
            pl.BlockSpec((_RB, 1), lambda i: (i, 0)),
        ],
        out_shape=[
            jax.ShapeDtypeStruct((N, H), jnp.float32),
            jax.ShapeDtypeStruct((N, 1), jnp.float32),
        ],
    )(x, w1, degp)


def _tc2_body(p_ref, h1p_ref, dis_ref, b1_ref, w2_ref, h2p_ref):
    dis = dis_ref[...]
    agg = p_ref[0] + p_ref[1] - h1p_ref[...]
    a1 = jnp.maximum(agg * dis + b1_ref[...], 0.0)
    h2 = jnp.dot(a1, w2_ref[...], preferred_element_type=jnp.float32)
    h2p_ref[...] = h2 * dis


def _tc2(p, h1p, dis, b1, w2):
    return pl.pallas_call(
        _tc2_body,
        grid=(_GRID,),
        in_specs=[
            pl.BlockSpec((NC, _RB, H), lambda i: (0, i, 0)),
            pl.BlockSpec((_RB, H), lambda i: (i, 0)),
            pl.BlockSpec((_RB, 1), lambda i: (i, 0)),
            pl.BlockSpec((1, H), lambda i: (0, 0)),
            pl.BlockSpec((H, C), lambda i: (0, 0)),
        ],
        out_specs=pl.BlockSpec((_RB, C), lambda i: (i, 0)),
        out_shape=jax.ShapeDtypeStruct((N, C), jnp.float32),
    )(p, h1p, dis, b1, w2)


def _tc3_body(q_ref, h2p_ref, dis_ref, b2_ref, out_ref):
    z = (q_ref[0] + q_ref[1] - h2p_ref[...]) * dis_ref[...] + b2_ref[...]
    m = jnp.max(z, axis=1, keepdims=True)
    s = jnp.sum(jnp.exp(z - m), axis=1, keepdims=True)
    out_ref[...] = z - m - jnp.log(s)


def _tc3(q, h2p, dis, b2):
    return pl.pallas_call(
        _tc3_body,
        grid=(_GRID,),
        in_specs=[
            pl.BlockSpec((NC, _RB, C), lambda i: (0, i, 0)),
            pl.BlockSpec((_RB, C), lambda i: (i, 0)),
            pl.BlockSpec((_RB, 1), lambda i: (i, 0)),
            pl.BlockSpec((1, C), lambda i: (0, 0)),
        ],
        out_specs=pl.BlockSpec((_RB, C), lambda i: (i, 0)),
        out_shape=jax.ShapeDtypeStruct((N, C), jnp.float32),
    )(q, h2p, dis, b2)


@jax.jit
def kernel(x, edge_index, W1, b1, W2, b2):
    degp = _deg_kernel(edge_index)
    h1p, dis = _tc1(x, W1, degp.reshape(NC, N, 1))
    p = _agg16(h1p, edge_index)
    h2p = _tc2(p, h1p, dis, b1.reshape(1, H), W2)
    q = _agg40(h2p, edge_index)
    return _tc3(q, h2p, dis, b2.reshape(1, C))


# speedup vs baseline: 62.6366x; 1.0928x over previous
import functools

import jax
import jax.numpy as jnp
from jax import lax
from jax.experimental import pallas as pl
from jax.experimental.pallas import tpu as pltpu
from jax.experimental.pallas import tpu_sc as plsc

N = 10000
E = 320000
D_IN = 128
H = 16
C = 40

NC = 2
NS = 16
NW = NC * NS
EPT = E // NW
CH = 80
NCH = EPT // CH
RPT = 632
RPT_LAST = N - 15 * RPT

_MESH = plsc.VectorSubcoreMesh(core_axis_name="c", subcore_axis_name="s")
_SC_PARAMS = pltpu.CompilerParams(use_tc_tiling_on_sc=False)


@functools.partial(
    pl.kernel,
    out_type=[jax.ShapeDtypeStruct((N,), jnp.float32),
              jax.ShapeDtypeStruct((N,), jnp.float32)],
    mesh=_MESH,
    compiler_params=_SC_PARAMS,
    scratch_types=[
        pltpu.VMEM((NCH, CH), jnp.int32),
        pltpu.VMEM((CH,), jnp.float32),
        pltpu.VMEM((1008,), jnp.float32),
        pltpu.VMEM_SHARED((N,), jnp.float32),
        pltpu.SemaphoreType.DMA,
        pltpu.SemaphoreType.DMA,
    ],
)
def _deg_kernel(edge_hbm, out0_hbm, out1_hbm, idx_v, ones_v, stage_v,
                deg_sh, sem, semi):
    cid = lax.axis_index("c")
    sid = lax.axis_index("s")
    wid = cid * NS + sid

    def fill_idx(j, _):
        pltpu.async_copy(edge_hbm.at[1, pl.ds(wid * EPT + j * CH, CH)],
                         idx_v.at[j], semi)
        return ()
    lax.fori_loop(0, NCH, fill_idx, ())

    def drain_idx(j, _):
        pltpu.make_async_copy(edge_hbm.at[1, pl.ds(wid * EPT + j * CH, CH)],
                              idx_v.at[j], semi).wait()
        return ()
    lax.fori_loop(0, NCH, drain_idx, ())

    def fill_ones(i, _):
        ones_v[pl.ds(i * 16, 16)] = jnp.ones((16,), jnp.float32)
        return ()
    lax.fori_loop(0, CH // 16, fill_ones, ())

    def fill_zeros(i, _):
        stage_v[pl.ds(i * 16, 16)] = jnp.zeros((16,), jnp.float32)
        return ()
    lax.fori_loop(0, 63, fill_zeros, ())

    @pl.when(sid < 10)
    def _():
        pltpu.sync_copy(stage_v.at[pl.ds(0, 1000)],
                        deg_sh.at[pl.ds(sid * 1000, 1000)])

    plsc.subcore_barrier()

    def body(j, _):
        pltpu.async_copy(ones_v, deg_sh.at[idx_v.at[j]], sem, add=True)
        return ()
    lax.fori_loop(0, NCH, body, ())

    def drain(j, _):
        pltpu.make_async_copy(ones_v, deg_sh.at[idx_v.at[j]], sem).wait()
        return ()
    lax.fori_loop(0, NCH, drain, ())

    plsc.subcore_barrier()

    @pl.when(sid < 10)
    def _():
        pltpu.sync_copy(deg_sh.at[pl.ds(sid * 1000, 1000)],
                        stage_v.at[pl.ds(0, 1000)])

        @pl.when(cid == 0)
        def _():
            pltpu.sync_copy(stage_v.at[pl.ds(0, 1000)],
                            out0_hbm.at[pl.ds(sid * 1000, 1000)])

        @pl.when(cid == 1)
        def _():
            pltpu.sync_copy(stage_v.at[pl.ds(0, 1000)],
                            out1_hbm.at[pl.ds(sid * 1000, 1000)])


def _make_agg_kernel(feat, table_in_spmem=False):
    extra = ([pltpu.VMEM_SHARED((N, feat), jnp.float32)]
             if table_in_spmem else [])
    @functools.partial(
        pl.kernel,
        out_type=jax.ShapeDtypeStruct((NC, N, feat), jnp.float32),
        mesh=_MESH,
        compiler_params=_SC_PARAMS,
        scratch_types=[
            pltpu.VMEM((NCH, CH), jnp.int32),
            pltpu.VMEM((NCH, CH), jnp.int32),
            [pltpu.VMEM((CH, feat), jnp.float32)] * 4,
            pltpu.VMEM((RPT, feat), jnp.float32),
            pltpu.VMEM_SHARED((N, feat), jnp.float32),
            [pltpu.SemaphoreType.DMA] * 4,
            [pltpu.SemaphoreType.DMA] * 4,
            pltpu.SemaphoreType.DMA,
        ] + extra,
    )
    def _agg(h_hbm, edge_hbm, out_hbm,
             src_v, dst_v, rows, stage_v, acc_sh, semg, sems, semi,
             *maybe_table):
        table = maybe_table[0] if maybe_table else h_hbm
        cid = lax.axis_index("c")
        sid = lax.axis_index("s")
        wid = cid * NS + sid

        def fill_idx(j, _):
            base = wid * EPT + j * CH
            pltpu.async_copy(edge_hbm.at[0, pl.ds(base, CH)],
                             src_v.at[j], semi)
            pltpu.async_copy(edge_hbm.at[1, pl.ds(base, CH)],
                             dst_v.at[j], semi)
            return ()
        lax.fori_loop(0, NCH, fill_idx, ())

        def drain_idx(j, _):
            base = wid * EPT + j * CH
            pltpu.make_async_copy(edge_hbm.at[0, pl.ds(base, CH)],
                                  src_v.at[j], semi).wait()
            pltpu.make_async_copy(edge_hbm.at[1, pl.ds(base, CH)],
                                  dst_v.at[j], semi).wait()
            return ()
        lax.fori_loop(0, NCH, drain_idx, ())
        @pl.when(sid < 15)
        def _():
            pltpu.sync_copy(h_hbm.at[pl.ds(sid * RPT, RPT)], stage_v)
            pltpu.sync_copy(stage_v, acc_sh.at[pl.ds(sid * RPT, RPT)])
            if table_in_spmem:
                pltpu.sync_copy(stage_v, table.at[pl.ds(sid * RPT, RPT)])

        @pl.when(sid == 15)
        def _():
            pltpu.sync_copy(h_hbm.at[pl.ds(15 * RPT, RPT_LAST)],
                            stage_v.at[pl.ds(0, RPT_LAST)])
            pltpu.sync_copy(stage_v.at[pl.ds(0, RPT_LAST)],
                            acc_sh.at[pl.ds(15 * RPT, RPT_LAST)])
            if table_in_spmem:
                pltpu.sync_copy(stage_v.at[pl.ds(0, RPT_LAST)],
                                table.at[pl.ds(15 * RPT, RPT_LAST)])

        plsc.subcore_barrier()

        def _gather(j, buf, sem):
            pltpu.async_copy(table.at[src_v.at[j]], buf, sem)

        def _scatter(j, buf, sem):
            pltpu.async_copy(buf, acc_sh.at[dst_v.at[j]], sem, add=True)

        def _wait_gather(j, buf, sem):
            pltpu.make_async_copy(table.at[src_v.at[j]], buf, sem).wait()

        def _wait_scatter(j, buf, sem):
            pltpu.make_async_copy(buf, acc_sh.at[dst_v.at[j]], sem).wait()

        for b in range(4):
            _gather(b, rows[b], semg[b])

        def group(g, _):
            for b in range(4):
                j = 4 * g + b
                _wait_gather(j, rows[b], semg[b])
                _scatter(j, rows[b], sems[b])

                @pl.when(jnp.logical_and(j >= 2, j <= NCH - 3))
                def _(j=j, b=b):
                    b2 = (b + 2) % 4
                    _wait_scatter(j - 2, rows[b2], sems[b2])
                    _gather(j + 2, rows[b2], semg[b2])
            return ()
        lax.fori_loop(0, NCH // 4, group, ())

        _wait_gather(NCH - 1, rows[0], semg[0])
        _scatter(NCH - 1, rows[0], sems[0])
        for b in range(4):
            _wait_scatter(0, rows[b], sems[b])

        plsc.subcore_barrier()

        @pl.when(sid < 15)
        def _():
            pltpu.sync_copy(acc_sh.at[pl.ds(sid * RPT, RPT)], stage_v)
            pltpu.sync_copy(stage_v,
                            out_hbm.at[cid, pl.ds(sid * RPT, RPT)])

        @pl.when(sid == 15)
        def _():
            pltpu.sync_copy(acc_sh.at[pl.ds(15 * RPT, RPT_LAST)],
                            stage_v.at[pl.ds(0, RPT_LAST)])
            pltpu.sync_copy(stage_v.at[pl.ds(0, RPT_LAST)],
                            out_hbm.at[cid, pl.ds(15 * RPT, RPT_LAST)])

    return _agg


_agg16 = _make_agg_kernel(H, table_in_spmem=True)
_agg40 = _make_agg_kernel(C, table_in_spmem=True)


_RB = 2048
_GRID = -(-N // _RB)


def _tc1_body(x_ref, w1_ref, deg0_ref, deg1_ref, h1p_ref, dis_ref):
    deg = deg0_ref[...] + deg1_ref[...] + 1.0
    dis = lax.rsqrt(deg).reshape(_RB, 1)
    h = jnp.dot(x_ref[...], w1_ref[...], preferred_element_type=jnp.float32)
    h1p_ref[...] = h * dis
    dis_ref[...] = dis


def _tc1(x, w1, deg0, deg1):
    return pl.pallas_call(
        _tc1_body,
        grid=(_GRID,),
        in_specs=[
            pl.BlockSpec((_RB, D_IN), lambda i: (i, 0)),
            pl.BlockSpec((D_IN, H), lambda i: (0, 0)),
            pl.BlockSpec((_RB,), lambda i: (i,)),
            pl.BlockSpec((_RB,), lambda i: (i,)),
        ],
        out_specs=[
            pl.BlockSpec((_RB, H), lambda i: (i, 0)),
            pl.BlockSpec((_RB, 1), lambda i: (i, 0)),
        ],
        out_shape=[
            jax.ShapeDtypeStruct((N, H), jnp.float32),
            jax.ShapeDtypeStruct((N, 1), jnp.float32),
        ],
    )(x, w1, deg0, deg1)


def _tc2_body(p_ref, h1p_ref, dis_ref, b1_ref, w2_ref, h2p_ref):
    dis = dis_ref[...]
    agg = p_ref[0] + p_ref[1] - h1p_ref[...]
    a1 = jnp.maximum(agg * dis + b1_ref[...], 0.0)
    h2 = jnp.dot(a1, w2_ref[...], preferred_element_type=jnp.float32)
    h2p_ref[...] = h2 * dis


def _tc2(p, h1p, dis, b1, w2):
    return pl.pallas_call(
        _tc2_body,
        grid=(_GRID,),
        in_specs=[
            pl.BlockSpec((NC, _RB, H), lambda i: (0, i, 0)),
            pl.BlockSpec((_RB, H), lambda i: (i, 0)),
            pl.BlockSpec((_RB, 1), lambda i: (i, 0)),
            pl.BlockSpec((1, H), lambda i: (0, 0)),
            pl.BlockSpec((H, C), lambda i: (0, 0)),
        ],
        out_specs=pl.BlockSpec((_RB, C), lambda i: (i, 0)),
        out_shape=jax.ShapeDtypeStruct((N, C), jnp.float32),
    )(p, h1p, dis, b1, w2)


def _tc3_body(q_ref, h2p_ref, dis_ref, b2_ref, out_ref):
    z = (q_ref[0] + q_ref[1] - h2p_ref[...]) * dis_ref[...] + b2_ref[...]
    m = jnp.max(z, axis=1, keepdims=True)
    s = jnp.sum(jnp.exp(z - m), axis=1, keepdims=True)
    out_ref[...] = z - m - jnp.log(s)


def _tc3(q, h2p, dis, b2):
    return pl.pallas_call(
        _tc3_body,
        grid=(_GRID,),
        in_specs=[
            pl.BlockSpec((NC, _RB, C), lambda i: (0, i, 0)),
            pl.BlockSpec((_RB, C), lambda i: (i, 0)),
            pl.BlockSpec((_RB, 1), lambda i: (i, 0)),
            pl.BlockSpec((1, C), lambda i: (0, 0)),
        ],
        out_specs=pl.BlockSpec((_RB, C), lambda i: (i, 0)),
        out_shape=jax.ShapeDtypeStruct((N, C), jnp.float32),
    )(q, h2p, dis, b2)


@jax.jit
def kernel(x, edge_index, W1, b1, W2, b2):
    deg0, deg1 = _deg_kernel(edge_index)
    h1p, dis = _tc1(x, W1, deg0, deg1)
    p = _agg16(h1p, edge_index)
    h2p = _tc2(p, h1p, dis, b1.reshape(1, H), W2)
    q = _agg40(h2p, edge_index)
    return _tc3(q, h2p, dis, b2.reshape(1, C))
